# baseline (speedup 1.0000x reference)
import functools
import jax
import jax.numpy as jnp
from jax import lax
from jax.experimental import pallas as pl
from jax.experimental.pallas import tpu as pltpu
from jax.experimental.pallas import tpu_sc as plsc

D_MODEL = 64
SCALE = 8.0
NC, NS, L = 2, 16, 16
NW = NC * NS
B_TOTAL = 4096 * 200
BPW = B_TOTAL // NW
CH = 1024
NCHUNK = BPW // CH
VECS_PER_ROW = D_MODEL // L


@functools.partial(
    pl.kernel,
    out_type=jax.ShapeDtypeStruct((B_TOTAL, D_MODEL), jnp.float32),
    mesh=plsc.VectorSubcoreMesh(core_axis_name="c", subcore_axis_name="s"),
    scratch_types=[
        pltpu.VMEM((CH,), jnp.int32),
        pltpu.VMEM((CH, D_MODEL), jnp.float32),
        pltpu.SemaphoreType.DMA,
    ],
    compiler_params=pltpu.CompilerParams(use_tc_tiling_on_sc=False),
)
def _embed_sc(idx_hbm, lut_hbm, out_hbm, idx_v, rows_v, sem):
    wid = lax.axis_index("s") * NC + lax.axis_index("c")
    base = wid * BPW

    def chunk_body(c, carry):
        off = base + c * CH
        pltpu.sync_copy(idx_hbm.at[pl.ds(off, CH)], idx_v)
        pltpu.async_copy(lut_hbm.at[idx_v], rows_v, sem).wait()

        def scale_body(i, carry2):
            for j in range(VECS_PER_ROW):
                sl = pl.ds(j * L, L)
                rows_v[i, sl] = rows_v[i, sl] * SCALE
            return carry2

        lax.fori_loop(0, CH, scale_body, 0)
        pltpu.sync_copy(rows_v, out_hbm.at[pl.ds(off, CH)])
        return carry

    lax.fori_loop(0, NCHUNK, chunk_body, 0)


def kernel(x, lut):
    out = _embed_sc(x.reshape(-1), lut)
    return out.reshape(x.shape + (lut.shape[1],))


# speedup vs baseline: 1.1080x; 1.1080x over previous
import functools
import jax
import jax.numpy as jnp
from jax import lax
from jax.experimental import pallas as pl
from jax.experimental.pallas import tpu as pltpu
from jax.experimental.pallas import tpu_sc as plsc

D_MODEL = 64
SCALE = 8.0
NC, NS, L = 2, 16, 16
NW = NC * NS
B_TOTAL = 4096 * 200
BPW = B_TOTAL // NW
CH = 400
NCHUNK = BPW // CH
VECS_PER_ROW = D_MODEL // L


@functools.partial(
    pl.kernel,
    out_type=jax.ShapeDtypeStruct((B_TOTAL, D_MODEL), jnp.float32),
    mesh=plsc.VectorSubcoreMesh(core_axis_name="c", subcore_axis_name="s"),
    scratch_types=[
        pltpu.VMEM((NCHUNK, CH), jnp.int32),
        pltpu.VMEM((CH, D_MODEL), jnp.float32),
        pltpu.VMEM((CH, D_MODEL), jnp.float32),
        pltpu.VMEM((CH, D_MODEL), jnp.float32),
        pltpu.VMEM((CH, D_MODEL), jnp.float32),
        pltpu.SemaphoreType.DMA,
        pltpu.SemaphoreType.DMA,
        pltpu.SemaphoreType.DMA,
        pltpu.SemaphoreType.DMA,
    ],
    compiler_params=pltpu.CompilerParams(use_tc_tiling_on_sc=False),
)
def _embed_sc(idx_hbm, lut_hbm, out_hbm, idx_all, gin0, gin1, gout0, gout1,
              sg0, sg1, sw0, sw1):
    wid = lax.axis_index("s") * NC + lax.axis_index("c")
    base = wid * BPW
    gin = (gin0, gin1)
    gout = (gout0, gout1)
    sg = (sg0, sg1)
    sw = (sw0, sw1)

    pltpu.sync_copy(idx_hbm.at[wid], idx_all)
    pltpu.async_copy(lut_hbm.at[idx_all.at[0]], gin0, sg0)
    pltpu.async_copy(lut_hbm.at[idx_all.at[1]], gin1, sg1)

    def group_body(k, carry):
        for b in range(2):
            c = 2 * k + b
            pltpu.make_async_copy(lut_hbm.at[idx_all.at[c]], gin[b], sg[b]).wait()

            def scale_body(i, carry2):
                for j in range(VECS_PER_ROW):
                    sl = pl.ds(j * L, L)
                    gout[b][i, sl] = gin[b][i, sl] * SCALE
                return carry2

            lax.fori_loop(0, CH, scale_body, 0)

            @pl.when(c + 2 < NCHUNK)
            def _():
                pltpu.async_copy(lut_hbm.at[idx_all.at[c + 2]], gin[b], sg[b])

            @pl.when(k >= 1)
            def _():
                pltpu.make_async_copy(
                    gout[b], out_hbm.at[pl.ds(base + c * CH, CH)], sw[b]
                ).wait()

            pltpu.async_copy(gout[b], out_hbm.at[pl.ds(base + c * CH, CH)], sw[b])
        return carry

    lax.fori_loop(0, NCHUNK // 2, group_body, 0)

    for b in range(2):
        c = NCHUNK - 2 + b
        pltpu.make_async_copy(
            gout[b], out_hbm.at[pl.ds(base + c * CH, CH)], sw[b]
        ).wait()


def kernel(x, lut):
    idx = x.reshape(NW, NCHUNK, CH)
    out = _embed_sc(idx, lut)
    return out.reshape(x.shape + (lut.shape[1],))
